# baseline (speedup 1.0000x reference)
import functools

import jax
import jax.numpy as jnp
from jax import lax
from jax.experimental import pallas as pl
from jax.experimental.pallas import tpu as pltpu
from jax.experimental.pallas import tpu_sc as plsc

NC = 2
NS = 16
NW = NC * NS
L = 16

CHUNK = 2048

_F16_SCALE = 2.0 ** 112
_F16_MASK = -0x70002000


def _pair_to_f32(lov):
    a = lax.shift_right_arithmetic(lax.shift_left(lov, 16), 3) & _F16_MASK
    b = lax.shift_right_arithmetic(lov, 3) & _F16_MASK
    lo = plsc.bitcast(a, jnp.float32) * jnp.float32(_F16_SCALE)
    hi = plsc.bitcast(b, jnp.float32) * jnp.float32(_F16_SCALE)
    return lo, hi


def _sampler_kernel(Hs, Ws, n_per_w, u_hbm, v_hbm, q_hbm,
                    o0_hbm, o1_hbm, o2_hbm,
                    ubuf, vbuf, wxbufs, wybufs, ibufs, qbufs, obufs, sems):
    wid = lax.axis_index("s") * NC + lax.axis_index("c")
    base_w = wid * n_per_w
    n_pairs = n_per_w // (2 * CHUNK)

    iota = lax.iota(jnp.int32, L)
    _COL0 = jnp.zeros((L,), jnp.int32)
    _COL1 = jnp.full((L,), 1, jnp.int32)
    _COL2 = jnp.full((L,), 2, jnp.int32)

    def stage_ab(base, s):
        pltpu.sync_copy(u_hbm.at[pl.ds(base, CHUNK)], ubuf)
        pltpu.sync_copy(v_hbm.at[pl.ds(base, CHUNK)], vbuf)

        def idx_body(t, _):
            sl = pl.ds(t * L, L)
            u = ubuf[sl]
            v = vbuf[sl]
            xs = u * jnp.float32(Ws)
            ys = v * jnp.float32(Hs)
            xi = xs.astype(jnp.int32)
            yi = ys.astype(jnp.int32)
            fx = xs - xi.astype(jnp.float32)
            fy = ys - yi.astype(jnp.float32)
            x0 = jnp.minimum(xi, Ws - 1)
            y0 = jnp.minimum(yi, Hs - 1)
            r0 = y0 * Ws
            r1 = jnp.minimum(r0 + Ws, (Hs - 1) * Ws)
            ibufs[s][0][sl] = r0 + x0
            ibufs[s][1][sl] = r1 + x0
            wxbufs[s][sl] = fx
            wybufs[s][sl] = fy
            return 0

        lax.fori_loop(0, CHUNK // L, idx_body, 0)

    def fire(s):
        for k in range(2):
            pltpu.async_copy(q_hbm.at[ibufs[s][k]], qbufs[s][k], sems[s])

    def drain(s):
        for k in range(2):
            pltpu.make_async_copy(
                q_hbm.at[ibufs[s][k]], qbufs[s][k], sems[s]).wait()

    def stage_de(base, s):
        def blend_body(t, _):
            sl = pl.ds(t * L, L)
            wx = wxbufs[s][sl]
            wy = wybufs[s][sl]
            prow = t * L + iota
            tex = []
            for k in range(2):
                qb = qbufs[s][k]
                w0 = plsc.load_gather(qb, [prow, _COL0])
                w1 = plsc.load_gather(qb, [prow, _COL1])
                w2 = plsc.load_gather(qb, [prow, _COL2])
                a0, a1 = _pair_to_f32(w0)
                b2, b2n = _pair_to_f32(w1)
                d0, d1 = _pair_to_f32(w2)
                tex.append([(a0, d0), (a1, d1), (b2, b2n)])
            for c in range(3):
                t00, t01 = tex[0][c]
                t10, t11 = tex[1][c]
                top = t00 + wx * (t01 - t00)
                bot = t10 + wx * (t11 - t10)
                obufs[c][sl] = top + wy * (bot - top)
            return 0

        lax.fori_loop(0, CHUNK // L, blend_body, 0)
        pltpu.sync_copy(obufs[0], o0_hbm.at[pl.ds(base, CHUNK)])
        pltpu.sync_copy(obufs[1], o1_hbm.at[pl.ds(base, CHUNK)])
        pltpu.sync_copy(obufs[2], o2_hbm.at[pl.ds(base, CHUNK)])

    stage_ab(base_w, 0)
    fire(0)

    def do_pair(gg, _):
        a = base_w + (2 * gg) * CHUNK
        b = a + CHUNK
        stage_ab(b, 1)
        fire(1)
        drain(0)
        stage_de(a, 0)

        @pl.when(gg < n_pairs - 1)
        def _():
            stage_ab(a + 2 * CHUNK, 0)
            fire(0)

        drain(1)
        stage_de(b, 1)
        return 0

    lax.fori_loop(0, n_pairs, do_pair, 0)


def kernel(x, data, resolution):
    del resolution
    Hs, Ws, C = data.shape
    N = x.shape[0]
    n_per_w = N // NW

    lo = lax.bitcast_convert_type(data[:, :, :2], jnp.int32)
    lon = jnp.concatenate([lo[:, 1:], lo[:, -1:]], axis=1)
    c2 = data[:, :, 2]
    c2n = jnp.concatenate([c2[:, 1:], c2[:, -1:]], axis=1)
    cp = lax.bitcast_convert_type(jnp.stack([c2, c2n], axis=-1), jnp.int32)
    q = jnp.stack([lo, cp, lon, cp], axis=-1).reshape(Hs * Ws, 4)
    u = x[:, 0]
    v = x[:, 1]

    mesh = plsc.VectorSubcoreMesh(core_axis_name="c", subcore_axis_name="s")
    sampler = pl.kernel(
        functools.partial(_sampler_kernel, Hs, Ws, n_per_w),
        out_type=[jax.ShapeDtypeStruct((N,), jnp.float32)] * 3,
        mesh=mesh,
        compiler_params=pltpu.CompilerParams(
            use_tc_tiling_on_sc=False, needs_layout_passes=False),
        scratch_types=[
            pltpu.VMEM((CHUNK,), jnp.float32),
            pltpu.VMEM((CHUNK,), jnp.float32),
            [pltpu.VMEM((CHUNK,), jnp.float32)] * 2,
            [pltpu.VMEM((CHUNK,), jnp.float32)] * 2,
            [[pltpu.VMEM((CHUNK,), jnp.int32)] * 2] * 2,
            [[pltpu.VMEM((CHUNK, 4), jnp.int32)] * 2] * 2,
            [pltpu.VMEM((CHUNK,), jnp.float32)] * 3,
            [pltpu.SemaphoreType.DMA] * 2,
        ],
    )
    o0, o1, o2 = sampler(u, v, q)
    return jnp.stack([o0, o1, o2], axis=1)


# speedup vs baseline: 18.0117x; 18.0117x over previous
import functools

import jax
import jax.numpy as jnp
from jax import lax
from jax.experimental import pallas as pl
from jax.experimental.pallas import tpu as pltpu
from jax.experimental.pallas import tpu_sc as plsc

NC = 2
NS = 16
NW = NC * NS
L = 16

CHUNK = 4096


_F16_SCALE = 2.0 ** 112


def _pair_to_f32(lov):
    a = lax.shift_right_arithmetic(lax.shift_left(lov, 16), 3) & (-0x70002000)
    b = lax.shift_right_arithmetic(lov, 3) & (-0x70002000)
    return plsc.bitcast(a, jnp.float32), plsc.bitcast(b, jnp.float32)


def _sampler_kernel(Hs, Ws, n_per_w, u_hbm, v_hbm, lo_hbm, cp_hbm,
                    o0_hbm, o1_hbm, o2_hbm,
                    ubuf, vbuf, wxbufs, wybufs, ibufs, lobufs, cpbufs,
                    obufs, sems):
    wid = lax.axis_index("s") * NC + lax.axis_index("c")
    base_w = wid * n_per_w
    n_pairs = n_per_w // (2 * CHUNK)

    def stage_ab(base, s):
        pltpu.sync_copy(u_hbm.at[pl.ds(base, CHUNK)], ubuf)
        pltpu.sync_copy(v_hbm.at[pl.ds(base, CHUNK)], vbuf)

        def idx_body(t, _):
            sl = pl.ds(t * L, L)
            u = ubuf[sl]
            v = vbuf[sl]
            xs = u * jnp.float32(Ws)
            ys = v * jnp.float32(Hs)
            xi = xs.astype(jnp.int32)
            yi = ys.astype(jnp.int32)
            fx = xs - xi.astype(jnp.float32)
            fy = ys - yi.astype(jnp.float32)
            x0 = jnp.minimum(xi, Ws - 1)
            y0 = jnp.minimum(yi, Hs - 1)
            x1 = jnp.minimum(x0 + 1, Ws - 1)
            y1 = jnp.minimum(y0 + 1, Hs - 1)
            r0 = y0 * Ws
            r1 = y1 * Ws
            ibufs[s][0][sl] = r0 + x0
            ibufs[s][1][sl] = r0 + x1
            ibufs[s][2][sl] = r1 + x0
            ibufs[s][3][sl] = r1 + x1
            wxbufs[s][sl] = fx
            wybufs[s][sl] = fy
            return 0

        lax.fori_loop(0, CHUNK // L, idx_body, 0, unroll=2)

    def fire(s):
        for k in range(4):
            pltpu.async_copy(lo_hbm.at[ibufs[s][k]], lobufs[s][k], sems[s])
        for k in range(2):
            pltpu.async_copy(cp_hbm.at[ibufs[s][2 * k]], cpbufs[s][k], sems[s])

    def drain(s):
        for k in range(4):
            pltpu.make_async_copy(
                lo_hbm.at[ibufs[s][k]], lobufs[s][k], sems[s]).wait()
        for k in range(2):
            pltpu.make_async_copy(
                cp_hbm.at[ibufs[s][2 * k]], cpbufs[s][k], sems[s]).wait()

    def stage_de(base, s):
        def blend_body(t, _):
            sl = pl.ds(t * L, L)
            wx = wxbufs[s][sl]
            wy = wybufs[s][sl]
            tex = []
            for k in range(4):
                c0, c1 = _pair_to_f32(lobufs[s][k][sl])
                tex.append([c0, c1, None])
            for k in range(2):
                c2a, c2b = _pair_to_f32(cpbufs[s][k][sl])
                tex[2 * k][2] = c2a
                tex[2 * k + 1][2] = c2b
            for c in range(3):
                top = tex[0][c] + wx * (tex[1][c] - tex[0][c])
                bot = tex[2][c] + wx * (tex[3][c] - tex[2][c])
                blended = top + wy * (bot - top)
                obufs[c][sl] = blended * jnp.float32(_F16_SCALE)
            return 0

        lax.fori_loop(0, CHUNK // L, blend_body, 0, unroll=2)
        pltpu.sync_copy(obufs[0], o0_hbm.at[pl.ds(base, CHUNK)])
        pltpu.sync_copy(obufs[1], o1_hbm.at[pl.ds(base, CHUNK)])
        pltpu.sync_copy(obufs[2], o2_hbm.at[pl.ds(base, CHUNK)])

    stage_ab(base_w, 0)
    fire(0)

    def do_pair(gg, _):
        a = base_w + (2 * gg) * CHUNK
        b = a + CHUNK
        stage_ab(b, 1)
        fire(1)
        drain(0)
        stage_de(a, 0)

        @pl.when(gg < n_pairs - 1)
        def _():
            stage_ab(a + 2 * CHUNK, 0)
            fire(0)

        drain(1)
        stage_de(b, 1)
        return 0

    lax.fori_loop(0, n_pairs, do_pair, 0)


def kernel(x, data, resolution):
    del resolution
    Hs, Ws, C = data.shape
    N = x.shape[0]
    n_per_w = N // NW

    lo = lax.bitcast_convert_type(data[:, :, :2], jnp.int32).reshape(Hs * Ws)
    c2 = data[:, :, 2]
    c2n = jnp.concatenate([c2[:, 1:], c2[:, -1:]], axis=1)
    cp = lax.bitcast_convert_type(
        jnp.stack([c2, c2n], axis=-1), jnp.int32).reshape(Hs * Ws)
    u = x[:, 0]
    v = x[:, 1]

    mesh = plsc.VectorSubcoreMesh(core_axis_name="c", subcore_axis_name="s")
    sampler = pl.kernel(
        functools.partial(_sampler_kernel, Hs, Ws, n_per_w),
        out_type=[jax.ShapeDtypeStruct((N,), jnp.float32)] * 3,
        mesh=mesh,
        compiler_params=pltpu.CompilerParams(
            use_tc_tiling_on_sc=False, needs_layout_passes=False),
        scratch_types=[
            pltpu.VMEM((CHUNK,), jnp.float32),
            pltpu.VMEM((CHUNK,), jnp.float32),
            [pltpu.VMEM((CHUNK,), jnp.float32)] * 2,
            [pltpu.VMEM((CHUNK,), jnp.float32)] * 2,
            [[pltpu.VMEM((CHUNK,), jnp.int32)] * 4] * 2,
            [[pltpu.VMEM((CHUNK,), jnp.int32)] * 4] * 2,
            [[pltpu.VMEM((CHUNK,), jnp.int32)] * 2] * 2,
            [pltpu.VMEM((CHUNK,), jnp.float32)] * 3,
            [pltpu.SemaphoreType.DMA] * 2,
        ],
    )
    o0, o1, o2 = sampler(u, v, lo, cp)
    return jnp.stack([o0, o1, o2], axis=1)
